# Initial kernel scaffold; baseline (speedup 1.0000x reference)
import functools

import jax
import jax.numpy as jnp
from jax import lax
from jax.experimental import pallas as pl
from jax.experimental.pallas import tpu as pltpu
from jax.experimental.pallas import tpu_sc as plsc

D = 128
NTILES = 32
NSUB = 16
NPAD = 10240
ROWS_PER_TILE = NPAD // NSUB
CHUNK = 128
CHUNKS_PER_TILE = 160
E2PAD = NTILES * CHUNKS_PER_TILE * CHUNK
IDX_ROWS = E2PAD // CHUNK


def _mesh():
    return plsc.VectorSubcoreMesh(core_axis_name="c", subcore_axis_name="s")


@functools.partial(
    pl.kernel,
    mesh=_mesh(),
    out_type=jax.ShapeDtypeStruct((2, NPAD), jnp.float32),
    scratch_types=[
        pltpu.VMEM((CHUNKS_PER_TILE, CHUNK), jnp.int32),
        pltpu.VMEM((CHUNK,), jnp.float32),
        pltpu.VMEM_SHARED((NPAD,), jnp.float32),
    ],
)
def _degree_kernel(sidx_hbm, zeros_hbm, deg_hbm, idx_v, ones_v, acc_sh):
    core = lax.axis_index("c")
    sid = lax.axis_index("s")
    wid = core * NSUB + sid

    @pl.loop(0, CHUNK // 16)
    def _(i):
        ones_v[pl.ds(i * 16, 16)] = jnp.full((16,), 1.0, jnp.float32)

    pltpu.sync_copy(zeros_hbm, acc_sh.at[pl.ds(sid * ROWS_PER_TILE, ROWS_PER_TILE)])
    pltpu.sync_copy(
        sidx_hbm.at[pl.ds(wid * CHUNKS_PER_TILE, CHUNKS_PER_TILE)], idx_v)
    plsc.subcore_barrier()

    @pl.loop(0, CHUNKS_PER_TILE)
    def _(j):
        pltpu.sync_copy(ones_v, acc_sh.at[idx_v.at[j]], add=True)

    plsc.subcore_barrier()
    pltpu.sync_copy(
        acc_sh.at[pl.ds(sid * ROWS_PER_TILE, ROWS_PER_TILE)],
        deg_hbm.at[core, pl.ds(sid * ROWS_PER_TILE, ROWS_PER_TILE)])


@functools.partial(
    pl.kernel,
    mesh=_mesh(),
    out_type=jax.ShapeDtypeStruct((2, NPAD, D), jnp.float32),
    scratch_types=[
        pltpu.VMEM((CHUNKS_PER_TILE, CHUNK), jnp.int32),
        pltpu.VMEM((CHUNKS_PER_TILE, CHUNK), jnp.int32),
        pltpu.VMEM((CHUNK, D), jnp.float32),
        pltpu.VMEM_SHARED((NPAD, D), jnp.float32),
        pltpu.SemaphoreType.DMA,
    ],
)
def _aggregate_kernel(y_hbm, sidx_hbm, didx_hbm, zeros_hbm, z_hbm,
                      sidx_v, didx_v, rows_v, acc_sh, sem):
    core = lax.axis_index("c")
    sid = lax.axis_index("s")
    wid = core * NSUB + sid

    pltpu.sync_copy(zeros_hbm, acc_sh.at[pl.ds(sid * ROWS_PER_TILE, ROWS_PER_TILE)])
    pltpu.sync_copy(
        sidx_hbm.at[pl.ds(wid * CHUNKS_PER_TILE, CHUNKS_PER_TILE)], sidx_v)
    pltpu.sync_copy(
        didx_hbm.at[pl.ds(wid * CHUNKS_PER_TILE, CHUNKS_PER_TILE)], didx_v)
    plsc.subcore_barrier()

    @pl.loop(0, CHUNKS_PER_TILE)
    def _(j):
        pltpu.async_copy(y_hbm.at[sidx_v.at[j]], rows_v, sem).wait()
        pltpu.sync_copy(rows_v, acc_sh.at[didx_v.at[j]], add=True)

    plsc.subcore_barrier()
    pltpu.sync_copy(
        acc_sh.at[pl.ds(sid * ROWS_PER_TILE, ROWS_PER_TILE)],
        z_hbm.at[core, pl.ds(sid * ROWS_PER_TILE, ROWS_PER_TILE)])


def _scale_body(deg2_ref, xp_ref, y_ref, dinv_ref):
    deg = deg2_ref[0, :] + deg2_ref[1, :]
    dinv = jnp.where(deg > 0.0, lax.rsqrt(jnp.maximum(deg, 1.0)), 0.0)
    dinv_ref[...] = dinv
    y_ref[...] = xp_ref[...] * dinv[:, None]


def _final_body(x_ref, z_ref, dinv_ref, w_ref, b_ref, o_ref):
    zsum = z_ref[0] + z_ref[1]
    xz = x_ref[...] + dinv_ref[...][:, None] * zsum
    r = lax.dot_general(
        xz, w_ref[...], (((1,), (1,)), ((), ())),
        preferred_element_type=jnp.float32, precision=lax.Precision.HIGHEST)
    o_ref[...] = jnp.maximum(r + b_ref[...][None, :], 0.0)


def kernel(x, edge_index, W, b):
    n = x.shape[0]
    n_edges = edge_index.shape[1]
    src = edge_index[0].astype(jnp.int32)
    dst = edge_index[1].astype(jnp.int32)
    pad = jnp.full((E2PAD - 2 * n_edges,), n, jnp.int32)
    sidx = jnp.concatenate([src, dst, pad]).reshape(IDX_ROWS, CHUNK)
    didx = jnp.concatenate([dst, src, pad]).reshape(IDX_ROWS, CHUNK)
    xp = jnp.pad(x, ((0, NPAD - n), (0, 0)))
    zeros1 = jnp.zeros((ROWS_PER_TILE,), jnp.float32)
    zeros2 = jnp.zeros((ROWS_PER_TILE, D), jnp.float32)

    deg2 = _degree_kernel(sidx, zeros1)

    y, dinv = pl.pallas_call(
        _scale_body,
        out_shape=[
            jax.ShapeDtypeStruct((NPAD, D), jnp.float32),
            jax.ShapeDtypeStruct((NPAD,), jnp.float32),
        ],
    )(deg2, xp)

    z2 = _aggregate_kernel(y, sidx, didx, zeros2)

    blk = 1000
    out = pl.pallas_call(
        _final_body,
        grid=(n // blk,),
        in_specs=[
            pl.BlockSpec((blk, D), lambda i: (i, 0)),
            pl.BlockSpec((2, blk, D), lambda i: (0, i, 0)),
            pl.BlockSpec((blk,), lambda i: (i,)),
            pl.BlockSpec((D, D), lambda i: (0, 0)),
            pl.BlockSpec((D,), lambda i: (0,)),
        ],
        out_specs=pl.BlockSpec((blk, D), lambda i: (i, 0)),
        out_shape=jax.ShapeDtypeStruct((n, D), jnp.float32),
    )(x, z2, dinv, W, b)
    return out


# speedup vs baseline: 7.8104x; 7.8104x over previous
import functools

import jax
import jax.numpy as jnp
from jax import lax
from jax.experimental import pallas as pl
from jax.experimental.pallas import tpu as pltpu
from jax.experimental.pallas import tpu_sc as plsc

D = 128
NTILES = 32
NSUB = 16
NPAD = 10240
ROWS_PER_TILE = NPAD // NSUB
CHUNK = 128
CHUNKS_PER_TILE = 160
E2PAD = NTILES * CHUNKS_PER_TILE * CHUNK
IDX_ROWS = E2PAD // CHUNK


def _mesh():
    return plsc.VectorSubcoreMesh(core_axis_name="c", subcore_axis_name="s")


@functools.partial(
    pl.kernel,
    mesh=_mesh(),
    out_type=jax.ShapeDtypeStruct((2, NPAD), jnp.float32),
    scratch_types=[
        pltpu.VMEM((CHUNKS_PER_TILE, CHUNK), jnp.int32),
        pltpu.VMEM((CHUNK,), jnp.float32),
        pltpu.VMEM_SHARED((NPAD,), jnp.float32),
    ],
)
def _degree_kernel(sidx_hbm, zeros_hbm, deg_hbm, idx_v, ones_v, acc_sh):
    core = lax.axis_index("c")
    sid = lax.axis_index("s")
    wid = core * NSUB + sid

    @pl.loop(0, CHUNK // 16)
    def _(i):
        ones_v[pl.ds(i * 16, 16)] = jnp.full((16,), 1.0, jnp.float32)

    pltpu.sync_copy(zeros_hbm, acc_sh.at[pl.ds(sid * ROWS_PER_TILE, ROWS_PER_TILE)])
    pltpu.sync_copy(
        sidx_hbm.at[pl.ds(wid * CHUNKS_PER_TILE, CHUNKS_PER_TILE)], idx_v)
    plsc.subcore_barrier()

    @pl.loop(0, CHUNKS_PER_TILE)
    def _(j):
        pltpu.sync_copy(ones_v, acc_sh.at[idx_v.at[j]], add=True)

    plsc.subcore_barrier()
    pltpu.sync_copy(
        acc_sh.at[pl.ds(sid * ROWS_PER_TILE, ROWS_PER_TILE)],
        deg_hbm.at[core, pl.ds(sid * ROWS_PER_TILE, ROWS_PER_TILE)])


GRP = 16


@functools.partial(
    pl.kernel,
    mesh=_mesh(),
    out_type=jax.ShapeDtypeStruct((2, NPAD, D), jnp.float32),
    scratch_types=[
        pltpu.VMEM((GRP, CHUNK), jnp.int32),
        pltpu.VMEM((GRP, CHUNK), jnp.int32),
        pltpu.VMEM((CHUNK, D), jnp.float32),
        pltpu.VMEM((CHUNK, D), jnp.float32),
        pltpu.VMEM_SHARED((NPAD, D), jnp.float32),
        pltpu.SemaphoreType.DMA,
        pltpu.SemaphoreType.DMA,
    ],
)
def _aggregate_kernel(y_hbm, sidx_hbm, didx_hbm, zeros_hbm, z_hbm,
                      sidx_v, didx_v, rows0, rows1, acc_sh, sem0, sem1):
    core = lax.axis_index("c")
    sid = lax.axis_index("s")
    wid = core * NSUB + sid

    pltpu.sync_copy(zeros_hbm, acc_sh.at[pl.ds(sid * ROWS_PER_TILE, ROWS_PER_TILE)])
    plsc.subcore_barrier()

    @pl.loop(0, CHUNKS_PER_TILE // GRP)
    def _(g):
        base = wid * CHUNKS_PER_TILE + g * GRP
        pltpu.sync_copy(sidx_hbm.at[pl.ds(base, GRP)], sidx_v)
        pltpu.sync_copy(didx_hbm.at[pl.ds(base, GRP)], didx_v)

        @pl.loop(0, GRP, step=2)
        def _(j):
            g0 = pltpu.async_copy(y_hbm.at[sidx_v.at[j]], rows0, sem0)
            g1 = pltpu.async_copy(y_hbm.at[sidx_v.at[j + 1]], rows1, sem1)
            g0.wait()
            pltpu.sync_copy(rows0, acc_sh.at[didx_v.at[j]], add=True)
            g1.wait()
            pltpu.sync_copy(rows1, acc_sh.at[didx_v.at[j + 1]], add=True)

    plsc.subcore_barrier()
    pltpu.sync_copy(
        acc_sh.at[pl.ds(sid * ROWS_PER_TILE, ROWS_PER_TILE)],
        z_hbm.at[core, pl.ds(sid * ROWS_PER_TILE, ROWS_PER_TILE)])


def _scale_body(deg2_ref, xp_ref, y_ref, dinv_ref):
    deg = deg2_ref[0, :] + deg2_ref[1, :]
    dinv = jnp.where(deg > 0.0, lax.rsqrt(jnp.maximum(deg, 1.0)), 0.0)
    dinv_ref[...] = dinv[:, None]
    y_ref[...] = xp_ref[...] * dinv[:, None]


def _final_body(x_ref, z_ref, dinv_ref, w_ref, b_ref, o_ref):
    zsum = z_ref[0] + z_ref[1]
    xz = x_ref[...] + dinv_ref[...] * zsum
    r = lax.dot_general(
        xz, w_ref[...], (((1,), (1,)), ((), ())),
        preferred_element_type=jnp.float32, precision=lax.Precision.HIGHEST)
    o_ref[...] = jnp.maximum(r + b_ref[...][None, :], 0.0)


def kernel(x, edge_index, W, b):
    n = x.shape[0]
    n_edges = edge_index.shape[1]
    src = edge_index[0].astype(jnp.int32)
    dst = edge_index[1].astype(jnp.int32)
    pad = jnp.full((E2PAD - 2 * n_edges,), n, jnp.int32)
    sidx = jnp.concatenate([src, dst, pad]).reshape(IDX_ROWS, CHUNK)
    didx = jnp.concatenate([dst, src, pad]).reshape(IDX_ROWS, CHUNK)
    xp = jnp.pad(x, ((0, NPAD - n), (0, 0)))
    zeros1 = jnp.zeros((ROWS_PER_TILE,), jnp.float32)
    zeros2 = jnp.zeros((ROWS_PER_TILE, D), jnp.float32)

    deg2 = _degree_kernel(sidx, zeros1)

    y, dinv = pl.pallas_call(
        _scale_body,
        out_shape=[
            jax.ShapeDtypeStruct((NPAD, D), jnp.float32),
            jax.ShapeDtypeStruct((NPAD, 1), jnp.float32),
        ],
    )(deg2, xp)

    z2 = _aggregate_kernel(y, sidx, didx, zeros2)

    blk = 1000
    out = pl.pallas_call(
        _final_body,
        grid=(n // blk,),
        in_specs=[
            pl.BlockSpec((blk, D), lambda i: (i, 0)),
            pl.BlockSpec((2, blk, D), lambda i: (0, i, 0)),
            pl.BlockSpec((blk, 1), lambda i: (i, 0)),
            pl.BlockSpec((D, D), lambda i: (0, 0)),
            pl.BlockSpec((D,), lambda i: (0,)),
        ],
        out_specs=pl.BlockSpec((blk, D), lambda i: (i, 0)),
        out_shape=jax.ShapeDtypeStruct((n, D), jnp.float32),
    )(x, z2, dinv, W, b)
    return out
